# baseline (speedup 1.0000x reference)
import functools

import jax
import jax.numpy as jnp
from jax import lax
from jax.experimental import pallas as pl
from jax.experimental.pallas import tpu as pltpu
from jax.experimental.pallas import tpu_sc as plsc

_K = 32
_BLOCK = 512


def _sq_tile(ph, plt):
    hh = jnp.sum(ph * ph, axis=1, keepdims=True)
    ll = jnp.sum(plt * plt, axis=0, keepdims=True)
    cross = jax.lax.dot_general(
        ph, plt, (((1,), (0,)), ((), ())),
        preferred_element_type=jnp.float32)
    return jnp.maximum((hh + ll) - 2.0 * cross, 0.0)


def _d2_body(ph_ref, plt_ref, out_ref):
    out_ref[...] = _sq_tile(ph_ref[...], plt_ref[...])


def _apply_body(ph_ref, plt_ref, x_ref, th_ref, out_ref):
    sq = _sq_tile(ph_ref[...], plt_ref[...])
    thresh = th_ref[...]
    w = jnp.where(sq <= thresh, 1.0 / jnp.maximum(sq, 1e-16), 0.0)
    x = x_ref[...]
    w_hi = w.astype(jnp.bfloat16)
    w_lo = (w - w_hi.astype(jnp.float32)).astype(jnp.bfloat16)
    x_hi = x.astype(jnp.bfloat16)
    x_lo = (x - x_hi.astype(jnp.float32)).astype(jnp.bfloat16)
    dims = (((1,), (0,)), ((), ()))
    num = (jax.lax.dot_general(w_hi, x_hi, dims,
                               preferred_element_type=jnp.float32)
           + jax.lax.dot_general(w_hi, x_lo, dims,
                                 preferred_element_type=jnp.float32)
           + jax.lax.dot_general(w_lo, x_hi, dims,
                                 preferred_element_type=jnp.float32))
    den = jnp.sum(w, axis=1, keepdims=True)
    out_ref[...] = num / den


def _merge_lo(a, b):
    lo = jnp.minimum(a, lax.rev(b, (0,)))
    return lax.sort(lo)


def _merge_full(a, b):
    br = lax.rev(b, (0,))
    lo = jnp.minimum(a, br)
    hi = jnp.maximum(a, br)
    return lax.sort(lo), lax.sort(hi)


def _make_sc_thresh(nh, nl):
    mesh = plsc.VectorSubcoreMesh(core_axis_name="c", subcore_axis_name="s")
    info = plsc.get_sparse_core_info()
    nw = info.num_cores * info.num_subcores
    rows_per_w = nh // nw

    @functools.partial(
        pl.kernel, mesh=mesh,
        out_type=jax.ShapeDtypeStruct((nh,), jnp.float32),
        compiler_params=pltpu.CompilerParams(needs_layout_passes=False),
        scratch_types=[
            pltpu.VMEM((nl,), jnp.float32),
            pltpu.VMEM((nl + 32,), jnp.float32),
            pltpu.VMEM((rows_per_w,), jnp.float32),
        ],
    )
    def sc_thresh(d2_hbm, out_hbm, rowbuf, cand, tbuf):
        wid = lax.axis_index("s") * info.num_cores + lax.axis_index("c")
        base = wid * rows_per_w
        inf16 = jnp.full((16,), jnp.float32(jnp.inf), jnp.float32)
        lane0 = lax.iota(jnp.int32, 16) == 0

        def row_body(r, _):
            pltpu.sync_copy(d2_hbm.at[base + r], rowbuf)

            def mina(i, carry):
                m0, m1 = carry
                m0 = jnp.minimum(m0, rowbuf[pl.ds(i * 32, 16)])
                m1 = jnp.minimum(m1, rowbuf[pl.ds(i * 32 + 16, 16)])
                return m0, m1
            m0, m1 = lax.fori_loop(0, nl // 32, mina, (inf16, inf16))
            u = lax.sort(jnp.maximum(m0, m1))[15]

            def compact(i, off):
                v = rowbuf[pl.ds(i * 16, 16)]
                msk = v <= u
                plsc.store_compressed(cand.at[pl.ds(off, 16)], v, mask=msk)
                return off + plsc.all_reduce_population_count(msk)[0]
            cnt = lax.fori_loop(0, nl // 16, compact, jnp.int32(0))
            cand[pl.ds(cnt, 16)] = inf16

            t0 = lax.sort(cand[pl.ds(0, 16)])
            t1 = lax.sort(cand[pl.ds(16, 16)])
            t0, t1 = _merge_full(t0, t1)

            def mrg(i, carry):
                t0, t1 = carry
                s = lax.sort(cand[pl.ds(i * 16, 16)])
                return _merge_full(t0, _merge_lo(t1, s))
            t0, t1 = lax.fori_loop(2, (cnt + 15) // 16, mrg, (t0, t1))
            plsc.store_scatter(tbuf, [jnp.full((16,), r, jnp.int32)],
                               jnp.full((16,), t1[15], jnp.float32),
                               mask=lane0)
            return 0

        lax.fori_loop(0, rows_per_w, row_body, 0)
        pltpu.sync_copy(tbuf, out_hbm.at[pl.ds(base, rows_per_w)])

    return sc_thresh


def kernel(x, pos_l, pos_h):
    nh = pos_h.shape[0]
    nl = pos_l.shape[0]
    d = x.shape[1]
    plt = pos_l.T

    sq = pl.pallas_call(
        _d2_body,
        grid=(nh // _BLOCK,),
        in_specs=[
            pl.BlockSpec((_BLOCK, 3), lambda i: (i, 0)),
            pl.BlockSpec((3, nl), lambda i: (0, 0)),
        ],
        out_specs=pl.BlockSpec((_BLOCK, nl), lambda i: (i, 0)),
        out_shape=jax.ShapeDtypeStruct((nh, nl), jnp.float32),
    )(pos_h, plt)

    thresh = _make_sc_thresh(nh, nl)(sq).reshape(nh, 1)

    return pl.pallas_call(
        _apply_body,
        grid=(nh // _BLOCK,),
        in_specs=[
            pl.BlockSpec((_BLOCK, 3), lambda i: (i, 0)),
            pl.BlockSpec((3, nl), lambda i: (0, 0)),
            pl.BlockSpec((nl, d), lambda i: (0, 0)),
            pl.BlockSpec((_BLOCK, 1), lambda i: (i, 0)),
        ],
        out_specs=pl.BlockSpec((_BLOCK, d), lambda i: (i, 0)),
        out_shape=jax.ShapeDtypeStruct((nh, d), x.dtype),
    )(pos_h, plt, x, thresh)


# speedup vs baseline: 1.6913x; 1.6913x over previous
import functools

import jax
import jax.numpy as jnp
from jax import lax
from jax.experimental import pallas as pl
from jax.experimental.pallas import tpu as pltpu
from jax.experimental.pallas import tpu_sc as plsc

_K = 32
_BLOCK = 512


def _sq_tile(ph, plt):
    hh = jnp.sum(ph * ph, axis=1, keepdims=True)
    ll = jnp.sum(plt * plt, axis=0, keepdims=True)
    cross = jax.lax.dot_general(
        ph, plt, (((1,), (0,)), ((), ())),
        preferred_element_type=jnp.float32)
    return jnp.maximum((hh + ll) - 2.0 * cross, 0.0)


def _d2_body(ph_ref, plt_ref, out_ref):
    out_ref[...] = _sq_tile(ph_ref[...], plt_ref[...])


def _apply_body(ph_ref, plt_ref, x_ref, th_ref, out_ref):
    sq = _sq_tile(ph_ref[...], plt_ref[...])
    thresh = th_ref[...]
    w = jnp.where(sq <= thresh, 1.0 / jnp.maximum(sq, 1e-16), 0.0)
    x = x_ref[...]
    w_hi = w.astype(jnp.bfloat16)
    w_lo = (w - w_hi.astype(jnp.float32)).astype(jnp.bfloat16)
    x_hi = x.astype(jnp.bfloat16)
    x_lo = (x - x_hi.astype(jnp.float32)).astype(jnp.bfloat16)
    dims = (((1,), (0,)), ((), ()))
    num = (jax.lax.dot_general(w_hi, x_hi, dims,
                               preferred_element_type=jnp.float32)
           + jax.lax.dot_general(w_hi, x_lo, dims,
                                 preferred_element_type=jnp.float32)
           + jax.lax.dot_general(w_lo, x_hi, dims,
                                 preferred_element_type=jnp.float32))
    den = jnp.sum(w, axis=1, keepdims=True)
    out_ref[...] = num / den


def _merge_lo(a, b):
    lo = jnp.minimum(a, lax.rev(b, (0,)))
    return lax.sort(lo)


def _merge_full(a, b):
    br = lax.rev(b, (0,))
    lo = jnp.minimum(a, br)
    hi = jnp.maximum(a, br)
    return lax.sort(lo), lax.sort(hi)


def _make_sc_thresh(nh, nl):
    mesh = plsc.VectorSubcoreMesh(core_axis_name="c", subcore_axis_name="s")
    info = plsc.get_sparse_core_info()
    nw = info.num_cores * info.num_subcores
    rows_per_w = nh // nw

    @functools.partial(
        pl.kernel, mesh=mesh,
        out_type=jax.ShapeDtypeStruct((nh,), jnp.float32),
        compiler_params=pltpu.CompilerParams(needs_layout_passes=False),
        scratch_types=[
            pltpu.VMEM((nl,), jnp.float32),
            pltpu.VMEM((nl + 32,), jnp.float32),
            pltpu.VMEM((rows_per_w,), jnp.float32),
        ],
    )
    def sc_thresh(d2_hbm, out_hbm, rowbuf, cand, tbuf):
        wid = lax.axis_index("s") * info.num_cores + lax.axis_index("c")
        base = wid * rows_per_w
        inf16 = jnp.full((16,), jnp.float32(jnp.inf), jnp.float32)
        lane0 = lax.iota(jnp.int32, 16) == 0

        def row_body(r, _):
            pltpu.sync_copy(d2_hbm.at[base + r], rowbuf)

            def mina(i, carry):
                m0, m1 = carry
                for s in range(4):
                    m0 = jnp.minimum(
                        m0, rowbuf[pl.ds(i * 128 + s * 32, 16)])
                    m1 = jnp.minimum(
                        m1, rowbuf[pl.ds(i * 128 + s * 32 + 16, 16)])
                return m0, m1
            m0, m1 = lax.fori_loop(0, nl // 128, mina, (inf16, inf16))
            u = lax.sort(jnp.maximum(m0, m1))[15]

            def compact(i, off):
                vs = [rowbuf[pl.ds((i * 4 + s) * 16, 16)] for s in range(4)]
                msks = [v <= u for v in vs]
                pcs = [plsc.all_reduce_population_count(m)[0] for m in msks]
                for s in range(4):
                    plsc.store_compressed(cand.at[pl.ds(off, 16)], vs[s],
                                          mask=msks[s])
                    off = off + pcs[s]
                return off
            cnt = lax.fori_loop(0, nl // 64, compact, jnp.int32(0))
            cand[pl.ds(cnt, 16)] = inf16

            t0 = lax.sort(cand[pl.ds(0, 16)])
            t1 = lax.sort(cand[pl.ds(16, 16)])
            t0, t1 = _merge_full(t0, t1)

            def mrg(i, carry):
                t0, t1 = carry
                s = lax.sort(cand[pl.ds(i * 16, 16)])
                return _merge_full(t0, _merge_lo(t1, s))
            t0, t1 = lax.fori_loop(2, (cnt + 15) // 16, mrg, (t0, t1))
            plsc.store_scatter(tbuf, [jnp.full((16,), r, jnp.int32)],
                               jnp.full((16,), t1[15], jnp.float32),
                               mask=lane0)
            return 0

        lax.fori_loop(0, rows_per_w, row_body, 0)
        pltpu.sync_copy(tbuf, out_hbm.at[pl.ds(base, rows_per_w)])

    return sc_thresh


def kernel(x, pos_l, pos_h):
    nh = pos_h.shape[0]
    nl = pos_l.shape[0]
    d = x.shape[1]
    plt = pos_l.T

    sq = pl.pallas_call(
        _d2_body,
        grid=(nh // _BLOCK,),
        in_specs=[
            pl.BlockSpec((_BLOCK, 3), lambda i: (i, 0)),
            pl.BlockSpec((3, nl), lambda i: (0, 0)),
        ],
        out_specs=pl.BlockSpec((_BLOCK, nl), lambda i: (i, 0)),
        out_shape=jax.ShapeDtypeStruct((nh, nl), jnp.float32),
    )(pos_h, plt)

    thresh = _make_sc_thresh(nh, nl)(sq).reshape(nh, 1)

    return pl.pallas_call(
        _apply_body,
        grid=(nh // _BLOCK,),
        in_specs=[
            pl.BlockSpec((_BLOCK, 3), lambda i: (i, 0)),
            pl.BlockSpec((3, nl), lambda i: (0, 0)),
            pl.BlockSpec((nl, d), lambda i: (0, 0)),
            pl.BlockSpec((_BLOCK, 1), lambda i: (i, 0)),
        ],
        out_specs=pl.BlockSpec((_BLOCK, d), lambda i: (i, 0)),
        out_shape=jax.ShapeDtypeStruct((nh, d), x.dtype),
    )(pos_h, plt, x, thresh)


# speedup vs baseline: 1.7841x; 1.0549x over previous
import functools

import jax
import jax.numpy as jnp
from jax import lax
from jax.experimental import pallas as pl
from jax.experimental.pallas import tpu as pltpu
from jax.experimental.pallas import tpu_sc as plsc

_K = 32
_BLOCK = 512


def _sq_tile(ph, plt):
    hh = jnp.sum(ph * ph, axis=1, keepdims=True)
    ll = jnp.sum(plt * plt, axis=0, keepdims=True)
    cross = jax.lax.dot_general(
        ph, plt, (((1,), (0,)), ((), ())),
        preferred_element_type=jnp.float32)
    return jnp.maximum((hh + ll) - 2.0 * cross, 0.0)


def _d2_body(ph_ref, plt_ref, out_ref):
    out_ref[...] = _sq_tile(ph_ref[...], plt_ref[...])


def _apply_body(ph_ref, plt_ref, x_ref, th_ref, out_ref):
    sq = _sq_tile(ph_ref[...], plt_ref[...])
    thresh = th_ref[...]
    w = jnp.where(sq <= thresh, 1.0 / jnp.maximum(sq, 1e-16), 0.0)
    x = x_ref[...]
    w_hi = w.astype(jnp.bfloat16)
    w_lo = (w - w_hi.astype(jnp.float32)).astype(jnp.bfloat16)
    x_hi = x.astype(jnp.bfloat16)
    x_lo = (x - x_hi.astype(jnp.float32)).astype(jnp.bfloat16)
    dims = (((1,), (0,)), ((), ()))
    num = (jax.lax.dot_general(w_hi, x_hi, dims,
                               preferred_element_type=jnp.float32)
           + jax.lax.dot_general(w_hi, x_lo, dims,
                                 preferred_element_type=jnp.float32)
           + jax.lax.dot_general(w_lo, x_hi, dims,
                                 preferred_element_type=jnp.float32))
    den = jnp.sum(w, axis=1, keepdims=True)
    out_ref[...] = num / den


def _merge_lo(a, b):
    lo = jnp.minimum(a, lax.rev(b, (0,)))
    return lax.sort(lo)


def _merge_full(a, b):
    br = lax.rev(b, (0,))
    lo = jnp.minimum(a, br)
    hi = jnp.maximum(a, br)
    return lax.sort(lo), lax.sort(hi)


def _make_sc_thresh(nh, nl):
    mesh = plsc.VectorSubcoreMesh(core_axis_name="c", subcore_axis_name="s")
    info = plsc.get_sparse_core_info()
    nw = info.num_cores * info.num_subcores
    rows_per_w = nh // nw

    @functools.partial(
        pl.kernel, mesh=mesh,
        out_type=jax.ShapeDtypeStruct((nh,), jnp.float32),
        compiler_params=pltpu.CompilerParams(needs_layout_passes=False),
        scratch_types=[
            pltpu.VMEM((8 * nl,), jnp.float32),
            pltpu.VMEM((nl + 32,), jnp.float32),
            pltpu.VMEM((rows_per_w,), jnp.float32),
        ],
    )
    def sc_thresh(d2_hbm, out_hbm, rowbatch, cand, tbuf):
        wid = lax.axis_index("s") * info.num_cores + lax.axis_index("c")
        base = wid * rows_per_w
        inf16 = jnp.full((16,), jnp.float32(jnp.inf), jnp.float32)
        lane0 = lax.iota(jnp.int32, 16) == 0

        def grp_body(g, _):
            pltpu.sync_copy(
                d2_hbm.at[pl.ds((base + g * 8) * nl, 8 * nl)], rowbatch)
            for j in range(8):
                _one_row(g * 8 + j, j * nl)
            return 0

        def _one_row(r, o):
            def mina(i, carry):
                m0, m1 = carry
                for s in range(4):
                    m0 = jnp.minimum(
                        m0, rowbatch[pl.ds(o + i * 128 + s * 32, 16)])
                    m1 = jnp.minimum(
                        m1, rowbatch[pl.ds(o + i * 128 + s * 32 + 16, 16)])
                return m0, m1
            m0, m1 = lax.fori_loop(0, nl // 128, mina, (inf16, inf16))
            u = lax.sort(jnp.maximum(m0, m1))[15]

            def compact(i, off):
                vs = [rowbatch[pl.ds(o + (i * 4 + s) * 16, 16)]
                      for s in range(4)]
                msks = [v <= u for v in vs]
                pcs = [plsc.all_reduce_population_count(m)[0] for m in msks]
                for s in range(4):
                    plsc.store_compressed(cand.at[pl.ds(off, 16)], vs[s],
                                          mask=msks[s])
                    off = off + pcs[s]
                return off
            cnt = lax.fori_loop(0, nl // 64, compact, jnp.int32(0))
            cand[pl.ds(cnt, 16)] = inf16

            t0 = lax.sort(cand[pl.ds(0, 16)])
            t1 = lax.sort(cand[pl.ds(16, 16)])
            t0, t1 = _merge_full(t0, t1)

            def mrg(i, carry):
                t0, t1 = carry
                s = lax.sort(cand[pl.ds(i * 16, 16)])
                return _merge_full(t0, _merge_lo(t1, s))
            t0, t1 = lax.fori_loop(2, (cnt + 15) // 16, mrg, (t0, t1))
            plsc.store_scatter(tbuf, [jnp.full((16,), r, jnp.int32)],
                               jnp.full((16,), t1[15], jnp.float32),
                               mask=lane0)

        lax.fori_loop(0, rows_per_w // 8, grp_body, 0)
        pltpu.sync_copy(tbuf, out_hbm.at[pl.ds(base, rows_per_w)])

    return sc_thresh


def kernel(x, pos_l, pos_h):
    nh = pos_h.shape[0]
    nl = pos_l.shape[0]
    d = x.shape[1]
    plt = pos_l.T

    sq = pl.pallas_call(
        _d2_body,
        grid=(nh // _BLOCK,),
        in_specs=[
            pl.BlockSpec((_BLOCK, 3), lambda i: (i, 0)),
            pl.BlockSpec((3, nl), lambda i: (0, 0)),
        ],
        out_specs=pl.BlockSpec((_BLOCK, nl), lambda i: (i, 0)),
        out_shape=jax.ShapeDtypeStruct((nh, nl), jnp.float32),
    )(pos_h, plt)

    thresh = _make_sc_thresh(nh, nl)(sq.reshape(-1)).reshape(nh, 1)

    return pl.pallas_call(
        _apply_body,
        grid=(nh // _BLOCK,),
        in_specs=[
            pl.BlockSpec((_BLOCK, 3), lambda i: (i, 0)),
            pl.BlockSpec((3, nl), lambda i: (0, 0)),
            pl.BlockSpec((nl, d), lambda i: (0, 0)),
            pl.BlockSpec((_BLOCK, 1), lambda i: (i, 0)),
        ],
        out_specs=pl.BlockSpec((_BLOCK, d), lambda i: (i, 0)),
        out_shape=jax.ShapeDtypeStruct((nh, d), x.dtype),
    )(pos_h, plt, x, thresh)


# speedup vs baseline: 1.9223x; 1.0774x over previous
import functools

import jax
import jax.numpy as jnp
from jax import lax
from jax.experimental import pallas as pl
from jax.experimental.pallas import tpu as pltpu
from jax.experimental.pallas import tpu_sc as plsc

_K = 32
_BLOCK = 512


def _sq_tile(ph, plt):
    hh = jnp.sum(ph * ph, axis=1, keepdims=True)
    ll = jnp.sum(plt * plt, axis=0, keepdims=True)
    cross = jax.lax.dot_general(
        ph, plt, (((1,), (0,)), ((), ())),
        preferred_element_type=jnp.float32)
    return jnp.maximum((hh + ll) - 2.0 * cross, 0.0)


def _d2_body(ph_ref, plt_ref, out_ref):
    out_ref[...] = _sq_tile(ph_ref[...], plt_ref[...])


def _apply_body(ph_ref, plt_ref, x_ref, th_ref, out_ref):
    sq = _sq_tile(ph_ref[...], plt_ref[...])
    thresh = th_ref[...]
    w = jnp.where(sq <= thresh, 1.0 / jnp.maximum(sq, 1e-16), 0.0)
    x = x_ref[...]
    w_hi = w.astype(jnp.bfloat16)
    w_lo = (w - w_hi.astype(jnp.float32)).astype(jnp.bfloat16)
    x_hi = x.astype(jnp.bfloat16)
    x_lo = (x - x_hi.astype(jnp.float32)).astype(jnp.bfloat16)
    dims = (((1,), (0,)), ((), ()))
    num = (jax.lax.dot_general(w_hi, x_hi, dims,
                               preferred_element_type=jnp.float32)
           + jax.lax.dot_general(w_hi, x_lo, dims,
                                 preferred_element_type=jnp.float32)
           + jax.lax.dot_general(w_lo, x_hi, dims,
                                 preferred_element_type=jnp.float32))
    den = jnp.sum(w, axis=1, keepdims=True)
    out_ref[...] = num / den


def _merge_lo(a, b):
    lo = jnp.minimum(a, lax.rev(b, (0,)))
    return lax.sort(lo)


def _merge_full(a, b):
    br = lax.rev(b, (0,))
    lo = jnp.minimum(a, br)
    hi = jnp.maximum(a, br)
    return lax.sort(lo), lax.sort(hi)


def _make_sc_thresh(nh, nl):
    mesh = plsc.VectorSubcoreMesh(core_axis_name="c", subcore_axis_name="s")
    info = plsc.get_sparse_core_info()
    nw = info.num_cores * info.num_subcores
    rows_per_w = nh // nw

    @functools.partial(
        pl.kernel, mesh=mesh,
        out_type=jax.ShapeDtypeStruct((nh,), jnp.float32),
        compiler_params=pltpu.CompilerParams(needs_layout_passes=False),
        scratch_types=[
            pltpu.VMEM((8 * nl,), jnp.float32),
            pltpu.VMEM((8 * nl,), jnp.float32),
            pltpu.VMEM((nl + 32,), jnp.float32),
            pltpu.VMEM((rows_per_w,), jnp.float32),
            pltpu.SemaphoreType.DMA,
            pltpu.SemaphoreType.DMA,
        ],
    )
    def sc_thresh(d2_hbm, out_hbm, bufa, bufb, cand, tbuf, sema, semb):
        wid = lax.axis_index("s") * info.num_cores + lax.axis_index("c")
        base = wid * rows_per_w
        inf16 = jnp.full((16,), jnp.float32(jnp.inf), jnp.float32)
        lane0 = lax.iota(jnp.int32, 16) == 0

        ngroups = rows_per_w // 8

        def copy_grp(g, buf, sem):
            pltpu.async_copy(
                d2_hbm.at[pl.ds((base + g * 8) * nl, 8 * nl)], buf, sem)

        def wait_grp(g, buf, sem):
            pltpu.make_async_copy(
                d2_hbm.at[pl.ds((base + g * 8) * nl, 8 * nl)],
                buf, sem).wait()

        def pair_body(k, _):
            ga = 2 * k
            gb = 2 * k + 1
            copy_grp(gb, bufb, semb)
            wait_grp(ga, bufa, sema)
            for j in range(8):
                _one_row(ga * 8 + j, j * nl, bufa)
            copy_grp((gb + 1) % ngroups, bufa, sema)
            wait_grp(gb, bufb, semb)
            for j in range(8):
                _one_row(gb * 8 + j, j * nl, bufb)
            return 0

        def _one_row(r, o, rowbatch):
            def mina(i, carry):
                m0, m1 = carry
                for s in range(4):
                    m0 = jnp.minimum(
                        m0, rowbatch[pl.ds(o + i * 128 + s * 32, 16)])
                    m1 = jnp.minimum(
                        m1, rowbatch[pl.ds(o + i * 128 + s * 32 + 16, 16)])
                return m0, m1
            m0, m1 = lax.fori_loop(0, nl // 128, mina, (inf16, inf16))
            u = lax.sort(jnp.maximum(m0, m1))[15]

            def compact(i, off):
                vs = [rowbatch[pl.ds(o + (i * 4 + s) * 16, 16)]
                      for s in range(4)]
                msks = [v <= u for v in vs]
                pcs = [plsc.all_reduce_population_count(m)[0] for m in msks]
                for s in range(4):
                    plsc.store_compressed(cand.at[pl.ds(off, 16)], vs[s],
                                          mask=msks[s])
                    off = off + pcs[s]
                return off
            cnt = lax.fori_loop(0, nl // 64, compact, jnp.int32(0))
            cand[pl.ds(cnt, 16)] = inf16

            t0 = lax.sort(cand[pl.ds(0, 16)])
            t1 = lax.sort(cand[pl.ds(16, 16)])
            t0, t1 = _merge_full(t0, t1)

            def mrg(i, carry):
                t0, t1 = carry
                s = lax.sort(cand[pl.ds(i * 16, 16)])
                return _merge_full(t0, _merge_lo(t1, s))
            t0, t1 = lax.fori_loop(2, (cnt + 15) // 16, mrg, (t0, t1))
            plsc.store_scatter(tbuf, [jnp.full((16,), r, jnp.int32)],
                               jnp.full((16,), t1[15], jnp.float32),
                               mask=lane0)

        copy_grp(0, bufa, sema)
        lax.fori_loop(0, ngroups // 2, pair_body, 0)
        wait_grp(0, bufa, sema)
        pltpu.sync_copy(tbuf, out_hbm.at[pl.ds(base, rows_per_w)])

    return sc_thresh


def kernel(x, pos_l, pos_h):
    nh = pos_h.shape[0]
    nl = pos_l.shape[0]
    d = x.shape[1]
    plt = pos_l.T

    sq = pl.pallas_call(
        _d2_body,
        grid=(nh // _BLOCK,),
        in_specs=[
            pl.BlockSpec((_BLOCK, 3), lambda i: (i, 0)),
            pl.BlockSpec((3, nl), lambda i: (0, 0)),
        ],
        out_specs=pl.BlockSpec((_BLOCK, nl), lambda i: (i, 0)),
        out_shape=jax.ShapeDtypeStruct((nh, nl), jnp.float32),
    )(pos_h, plt)

    thresh = _make_sc_thresh(nh, nl)(sq.reshape(-1)).reshape(nh, 1)

    return pl.pallas_call(
        _apply_body,
        grid=(nh // _BLOCK,),
        in_specs=[
            pl.BlockSpec((_BLOCK, 3), lambda i: (i, 0)),
            pl.BlockSpec((3, nl), lambda i: (0, 0)),
            pl.BlockSpec((nl, d), lambda i: (0, 0)),
            pl.BlockSpec((_BLOCK, 1), lambda i: (i, 0)),
        ],
        out_specs=pl.BlockSpec((_BLOCK, d), lambda i: (i, 0)),
        out_shape=jax.ShapeDtypeStruct((nh, d), x.dtype),
    )(pos_h, plt, x, thresh)
